# baseline (speedup 1.0000x reference)
import jax
import jax.numpy as jnp
from jax.experimental import pallas as pl
from jax.experimental.pallas import tpu as pltpu

_LANE = 128
_BLOCK_LANES = 2048
_CHUNK = 128


def _round_up(x, m):
    return (x + m - 1) // m * m


def _mlp_kernel(xt_ref, w1c_ref, b1c_ref, w2c_ref, b2c_ref, out_ref):
    n_states = xt_ref.shape[0]
    n_act = out_ref.shape[0]
    L = xt_ref.shape[1]

    for c in range(L // _CHUNK):
        sl = slice(c * _CHUNK, (c + 1) * _CHUNK)
        xc = xt_ref[:, sl]
        h = b1c_ref[...]
        for k in range(n_states):
            h = h + w1c_ref[k] * xc[k : k + 1, :]
        h = jnp.maximum(h, 0.0)
        for a in range(n_act):
            ya = jnp.sum(h * w2c_ref[a], axis=0, keepdims=True)
            out_ref[a : a + 1, sl] = ya + b2c_ref[a : a + 1, :]


def kernel(x, w1p, b1p, w2p, b2p):
    B, n_states = x.shape
    hp = w1p.shape[1]
    n_act = 2

    xt = x.T
    bp = _round_up(B, _BLOCK_LANES)
    if bp != B:
        xt = jnp.pad(xt, ((0, 0), (0, bp - B)))

    w1c = jnp.broadcast_to(w1p[:, :, None], (n_states, hp, _CHUNK))
    b1c = jnp.broadcast_to(b1p[0][:, None], (hp, _CHUNK))
    w2c = jnp.broadcast_to(w2p[:, :n_act].T[:, :, None], (n_act, hp, _CHUNK))
    b2c = jnp.broadcast_to(b2p[0, :n_act][:, None], (n_act, _CHUNK))

    grid = bp // _BLOCK_LANES
    yt = pl.pallas_call(
        _mlp_kernel,
        out_shape=jax.ShapeDtypeStruct((n_act, bp), jnp.float32),
        grid=(grid,),
        in_specs=[
            pl.BlockSpec((n_states, _BLOCK_LANES), lambda i: (0, i)),
            pl.BlockSpec((n_states, hp, _CHUNK), lambda i: (0, 0, 0)),
            pl.BlockSpec((hp, _CHUNK), lambda i: (0, 0)),
            pl.BlockSpec((n_act, hp, _CHUNK), lambda i: (0, 0, 0)),
            pl.BlockSpec((n_act, _CHUNK), lambda i: (0, 0)),
        ],
        out_specs=pl.BlockSpec((n_act, _BLOCK_LANES), lambda i: (0, i)),
        compiler_params=pltpu.CompilerParams(
            dimension_semantics=("parallel",),
        ),
        cost_estimate=pl.CostEstimate(
            flops=2 * bp * (n_states * hp + hp * n_act),
            transcendentals=0,
            bytes_accessed=4 * (bp * n_states + bp * n_act),
        ),
    )(xt, w1c, b1c, w2c, b2c)

    return yt[:, :B].T


# speedup vs baseline: 1.3367x; 1.3367x over previous
import jax
import jax.numpy as jnp
from jax.experimental import pallas as pl
from jax.experimental.pallas import tpu as pltpu

_LANE = 128
_BLOCK_LANES = 2048
_CHUNK = 128


def _round_up(x, m):
    return (x + m - 1) // m * m


def _mlp_kernel(xa_ref, w1a_ref, w2c_ref, b2c_ref, out_ref):
    n_act = out_ref.shape[0]
    L = xa_ref.shape[1]
    w1a = w1a_ref[...]

    for c in range(L // _CHUNK):
        sl = slice(c * _CHUNK, (c + 1) * _CHUNK)
        xc = xa_ref[:, sl]
        h = jnp.dot(w1a, xc, preferred_element_type=jnp.float32)
        h = jnp.maximum(h, 0.0)
        for a in range(n_act):
            ya = jnp.sum(h * w2c_ref[a], axis=0, keepdims=True)
            out_ref[a : a + 1, sl] = ya + b2c_ref[a : a + 1, :]


def kernel(x, w1p, b1p, w2p, b2p):
    B, n_states = x.shape
    hp = w1p.shape[1]
    n_act = 2

    xa = jnp.concatenate([x.T, jnp.ones((1, B), x.dtype)], axis=0)
    bp = _round_up(B, _BLOCK_LANES)
    if bp != B:
        xa = jnp.pad(xa, ((0, 0), (0, bp - B)))

    w1a = jnp.concatenate([w1p.T, b1p.reshape(hp, 1)], axis=1)
    w2c = jnp.broadcast_to(w2p[:, :n_act].T[:, :, None], (n_act, hp, _CHUNK))
    b2c = jnp.broadcast_to(b2p[0, :n_act][:, None], (n_act, _CHUNK))

    grid = bp // _BLOCK_LANES
    yt = pl.pallas_call(
        _mlp_kernel,
        out_shape=jax.ShapeDtypeStruct((n_act, bp), jnp.float32),
        grid=(grid,),
        in_specs=[
            pl.BlockSpec((n_states + 1, _BLOCK_LANES), lambda i: (0, i)),
            pl.BlockSpec((hp, n_states + 1), lambda i: (0, 0)),
            pl.BlockSpec((n_act, hp, _CHUNK), lambda i: (0, 0, 0)),
            pl.BlockSpec((n_act, _CHUNK), lambda i: (0, 0)),
        ],
        out_specs=pl.BlockSpec((n_act, _BLOCK_LANES), lambda i: (0, i)),
        compiler_params=pltpu.CompilerParams(
            dimension_semantics=("parallel",),
        ),
        cost_estimate=pl.CostEstimate(
            flops=2 * bp * ((n_states + 1) * hp + hp * n_act),
            transcendentals=0,
            bytes_accessed=4 * (bp * (n_states + 1) + bp * n_act),
        ),
    )(xa, w1a, w2c, b2c)

    return yt[:, :B].T


# speedup vs baseline: 2.0981x; 1.5696x over previous
import jax
import jax.numpy as jnp
from jax.experimental import pallas as pl
from jax.experimental.pallas import tpu as pltpu

_LANE = 128
_BLOCK_LANES = 32768
_CHUNK = 128


def _round_up(x, m):
    return (x + m - 1) // m * m


def _mlp_kernel(xa_ref, w1a_ref, w2c_ref, b2c_ref, out_ref):
    n_act = out_ref.shape[0]
    L = xa_ref.shape[1]
    w1a = w1a_ref[...]

    for c in range(L // _CHUNK):
        sl = slice(c * _CHUNK, (c + 1) * _CHUNK)
        xc = xa_ref[:, sl]
        h = jnp.dot(w1a, xc, preferred_element_type=jnp.float32)
        h = jnp.maximum(h, 0.0)
        hh = pltpu.repeat(h, n_act, axis=1)
        ya = jnp.sum(hh * w2c_ref[...], axis=0, keepdims=True)
        yb = ya + b2c_ref[...]
        for a in range(n_act):
            out_ref[a : a + 1, sl] = yb[:, a * _CHUNK : (a + 1) * _CHUNK]


def kernel(x, w1p, b1p, w2p, b2p):
    B, n_states = x.shape
    hp = w1p.shape[1]
    n_act = 2

    xa = jnp.concatenate([x.T, jnp.ones((1, B), x.dtype)], axis=0)
    bp = _round_up(B, _BLOCK_LANES)
    if bp != B:
        xa = jnp.pad(xa, ((0, 0), (0, bp - B)))

    w1a = jnp.concatenate([w1p.T, b1p.reshape(hp, 1)], axis=1)
    w2c = jnp.broadcast_to(
        w2p[:, :n_act].T[:, :, None], (n_act, hp, _CHUNK)
    ).transpose(1, 0, 2).reshape(hp, n_act * _CHUNK)
    b2c = jnp.broadcast_to(
        b2p[0, :n_act][:, None], (n_act, _CHUNK)
    ).reshape(1, n_act * _CHUNK)

    grid = bp // _BLOCK_LANES
    yt = pl.pallas_call(
        _mlp_kernel,
        out_shape=jax.ShapeDtypeStruct((n_act, bp), jnp.float32),
        grid=(grid,),
        in_specs=[
            pl.BlockSpec((n_states + 1, _BLOCK_LANES), lambda i: (0, i)),
            pl.BlockSpec((hp, n_states + 1), lambda i: (0, 0)),
            pl.BlockSpec((hp, n_act * _CHUNK), lambda i: (0, 0)),
            pl.BlockSpec((1, n_act * _CHUNK), lambda i: (0, 0)),
        ],
        out_specs=pl.BlockSpec((n_act, _BLOCK_LANES), lambda i: (0, i)),
        compiler_params=pltpu.CompilerParams(
            dimension_semantics=("parallel",),
        ),
        cost_estimate=pl.CostEstimate(
            flops=2 * bp * ((n_states + 1) * hp + hp * n_act),
            transcendentals=0,
            bytes_accessed=4 * (bp * (n_states + 1) + bp * n_act),
        ),
    )(xa, w1a, w2c, b2c)

    return yt[:, :B].T


# speedup vs baseline: 2.9607x; 1.4111x over previous
import jax
import jax.numpy as jnp
from jax.experimental import pallas as pl
from jax.experimental.pallas import tpu as pltpu

_LANE = 128
_BLOCK_LANES = 32768
_CHUNK = 128


def _round_up(x, m):
    return (x + m - 1) // m * m


def _mlp_kernel(xa_ref, w1a_ref, w2c_ref, b2c_ref, out_ref):
    n_act = out_ref.shape[0]
    L = xa_ref.shape[1]
    w1a = w1a_ref[...]

    for c in range(L // _CHUNK):
        sl = slice(c * _CHUNK, (c + 1) * _CHUNK)
        xc = xa_ref[:, sl]
        h = jnp.dot(w1a, xc, preferred_element_type=jnp.float32)
        hb = jnp.maximum(h.astype(jnp.bfloat16), jnp.bfloat16(0))
        for q in range(_CHUNK // _LANE):
            hq = hb[:, q * _LANE : (q + 1) * _LANE]
            hh = pltpu.repeat(hq, n_act, axis=1)
            prod = hh * w2c_ref[...]
            hp_rows = prod.shape[0]
            while hp_rows > 16:
                hp_rows //= 2
                prod = prod[:hp_rows, :] + prod[hp_rows:, :]
            ya = jnp.sum(prod.astype(jnp.float32), axis=0, keepdims=True)
            yb = ya + b2c_ref[...]
            base = c * _CHUNK + q * _LANE
            for a in range(n_act):
                out_ref[a : a + 1, base : base + _LANE] = (
                    yb[:, a * _LANE : (a + 1) * _LANE]
                )


def kernel(x, w1p, b1p, w2p, b2p):
    B, n_states = x.shape
    hp = w1p.shape[1]
    n_act = 2

    xa = jnp.concatenate([x.T, jnp.ones((1, B), x.dtype)], axis=0)
    bp = _round_up(B, _BLOCK_LANES)
    if bp != B:
        xa = jnp.pad(xa, ((0, 0), (0, bp - B)))

    w1a = jnp.concatenate([w1p.T, b1p.reshape(hp, 1)], axis=1)
    w2c = jnp.broadcast_to(
        w2p[:, :n_act].T[:, :, None], (n_act, hp, _LANE)
    ).transpose(1, 0, 2).reshape(hp, n_act * _LANE).astype(jnp.bfloat16)
    b2c = jnp.broadcast_to(
        b2p[0, :n_act][:, None], (n_act, _LANE)
    ).reshape(1, n_act * _LANE)

    grid = bp // _BLOCK_LANES
    yt = pl.pallas_call(
        _mlp_kernel,
        out_shape=jax.ShapeDtypeStruct((n_act, bp), jnp.float32),
        grid=(grid,),
        in_specs=[
            pl.BlockSpec((n_states + 1, _BLOCK_LANES), lambda i: (0, i)),
            pl.BlockSpec((hp, n_states + 1), lambda i: (0, 0)),
            pl.BlockSpec((hp, n_act * _LANE), lambda i: (0, 0)),
            pl.BlockSpec((1, n_act * _LANE), lambda i: (0, 0)),
        ],
        out_specs=pl.BlockSpec((n_act, _BLOCK_LANES), lambda i: (0, i)),
        compiler_params=pltpu.CompilerParams(
            dimension_semantics=("parallel",),
        ),
        cost_estimate=pl.CostEstimate(
            flops=2 * bp * ((n_states + 1) * hp + hp * n_act),
            transcendentals=0,
            bytes_accessed=4 * (bp * (n_states + 1) + bp * n_act),
        ),
    )(xa, w1a, w2c, b2c)

    return yt[:, :B].T
